# baseline (speedup 1.0000x reference)
import functools

import jax
import jax.numpy as jnp
from jax import lax
from jax.experimental import pallas as pl
from jax.experimental.pallas import tpu as pltpu
from jax.experimental.pallas import tpu_sc as plsc

N_NODES = 10000
N_PAD = 10240
JUNK_ROW = N_NODES
BLK = 512
CH = 128
NB = 2
SB = 6
N_TILES = 32
RPT = N_PAD // 16

_MESH = dict(core_axis_name="c", subcore_axis_name="s")
_SC_PARAMS = dict(
    compiler_params=pltpu.CompilerParams(use_tc_tiling_on_sc=False))


def _fill(ref, rows, width, vec):
  for r in range(rows):
    for q in range(width // 16):
      ref[r, pl.ds(q * 16, 16)] = vec


def _tile_span(nck, wid):
  base, rem = nck // N_TILES, nck % N_TILES
  chunk0 = wid * base + jnp.minimum(wid, rem)
  nch_w = base + jnp.where(wid < rem, 1, 0)
  return base, chunk0, nch_w


def _sc_deg(nck):
  @functools.partial(
      pl.kernel,
      mesh=plsc.VectorSubcoreMesh(**_MESH),
      out_type=jax.ShapeDtypeStruct((2, N_PAD, 16), jnp.float32),
      scratch_types=[
          pltpu.VMEM_SHARED((N_PAD, 16), jnp.float32),
          pltpu.VMEM((SB, CH), jnp.int32),
          pltpu.VMEM((CH, 16), jnp.float32),
          pltpu.VMEM((64, 16), jnp.float32),
          pltpu.SemaphoreType.DMA,
      ],
      **_SC_PARAMS,
  )
  def k(dst_hbm, out_hbm, acc, dst_v, ones_v, zblk, sem):
    c = lax.axis_index("c")
    s = lax.axis_index("s")
    wid = s * 2 + c
    base, chunk0, nch_w = _tile_span(nck, wid)
    _fill(ones_v, CH, 16, jnp.ones((16,), jnp.float32))
    _fill(zblk, 64, 16, jnp.zeros((16,), jnp.float32))
    row0 = s * RPT

    def zbody(j, carry):
      pltpu.sync_copy(zblk, acc.at[pl.ds(row0 + j * 64, 64)])
      return carry

    lax.fori_loop(0, RPT // 64, zbody, 0)
    plsc.subcore_barrier()

    def sblock(sb, carry):
      pltpu.sync_copy(dst_hbm.at[pl.ds(chunk0 + sb * SB, SB)], dst_v)
      hs = [pltpu.async_copy(ones_v, acc.at[dst_v.at[b]], sem, add=True)
            for b in range(SB)]
      for h in hs:
        h.wait()
      return carry

    lax.fori_loop(0, base // SB, sblock, 0)

    def tailc(t, carry):
      pltpu.sync_copy(dst_hbm.at[pl.ds(chunk0 + t, 1)], dst_v.at[pl.ds(0, 1)])
      pltpu.async_copy(ones_v, acc.at[dst_v.at[0]], sem, add=True).wait()
      return carry

    lax.fori_loop((base // SB) * SB, nch_w, tailc, 0)
    plsc.subcore_barrier()
    pltpu.sync_copy(acc.at[pl.ds(row0, RPT)], out_hbm.at[c, pl.ds(row0, RPT)])

  return k


def _sc_agg(D, nck):
  @functools.partial(
      pl.kernel,
      mesh=plsc.VectorSubcoreMesh(**_MESH),
      out_type=jax.ShapeDtypeStruct((2, N_PAD, D), jnp.float32),
      scratch_types=[
          pltpu.VMEM_SHARED((N_PAD, D), jnp.float32),
          pltpu.VMEM((SB, CH), jnp.int32),
          pltpu.VMEM((SB, CH), jnp.int32),
      ] + [pltpu.VMEM((CH, D), jnp.float32) for _ in range(NB)] + [
          pltpu.VMEM((16, D), jnp.float32),
          pltpu.SemaphoreType.DMA,
          pltpu.SemaphoreType.DMA,
      ],
      **_SC_PARAMS,
  )
  def k(y_hbm, src_hbm, dst_hbm, out_hbm, acc, src_v, dst_v,
        *bufs_zblk_sems):
    rows = list(bufs_zblk_sems[:NB])
    zblk, gsem, ssem = bufs_zblk_sems[NB:]
    c = lax.axis_index("c")
    s = lax.axis_index("s")
    wid = s * 2 + c
    base, chunk0, nch_w = _tile_span(nck, wid)
    _fill(zblk, 16, D, jnp.zeros((16,), jnp.float32))
    row0 = s * RPT

    def zbody(j, carry):
      pltpu.sync_copy(zblk, acc.at[pl.ds(row0 + j * 16, 16)])
      return carry

    lax.fori_loop(0, RPT // 16, zbody, 0)
    plsc.subcore_barrier()

    def sblock(sb, carry):
      cb = chunk0 + sb * SB
      pltpu.sync_copy(src_hbm.at[pl.ds(cb, SB)], src_v)
      pltpu.sync_copy(dst_hbm.at[pl.ds(cb, SB)], dst_v)

      def group(g, carry2):
        j0 = g * NB
        ghs = [pltpu.async_copy(y_hbm.at[src_v.at[j0 + b]], rows[b], gsem)
               for b in range(NB)]
        shs = []
        for b in range(NB):
          ghs[b].wait()
          shs.append(pltpu.async_copy(rows[b], acc.at[dst_v.at[j0 + b]], ssem,
                                      add=True))
        for h in shs:
          h.wait()
        return carry2

      lax.fori_loop(0, SB // NB, group, 0)
      return carry

    lax.fori_loop(0, base // SB, sblock, 0)

    def tailc(t, carry):
      pltpu.sync_copy(src_hbm.at[pl.ds(chunk0 + t, 1)], src_v.at[pl.ds(0, 1)])
      pltpu.sync_copy(dst_hbm.at[pl.ds(chunk0 + t, 1)], dst_v.at[pl.ds(0, 1)])
      pltpu.async_copy(y_hbm.at[src_v.at[0]], rows[0], gsem).wait()
      pltpu.async_copy(rows[0], acc.at[dst_v.at[0]], ssem, add=True).wait()
      return carry

    lax.fori_loop((base // SB) * SB, nch_w, tailc, 0)
    plsc.subcore_barrier()
    pltpu.sync_copy(acc.at[pl.ds(row0, RPT)], out_hbm.at[c, pl.ds(row0, RPT)])

  return k


def _mm_body(x_ref, w_ref, o_ref):
  o_ref[...] = jnp.dot(x_ref[...], w_ref[...], preferred_element_type=jnp.float32)


def _tc_matmul(x, w):
  kdim = x.shape[1]
  dout = w.shape[1]
  return pl.pallas_call(
      _mm_body,
      grid=(N_PAD // BLK,),
      in_specs=[
          pl.BlockSpec((BLK, kdim), lambda i: (i, 0)),
          pl.BlockSpec((kdim, dout), lambda i: (0, 0)),
      ],
      out_specs=pl.BlockSpec((BLK, dout), lambda i: (i, 0)),
      out_shape=jax.ShapeDtypeStruct((N_PAD, dout), jnp.float32),
  )(x, w)


def _inv_deg(d0, d1):
  deg = jnp.maximum(d0[0] + d1[0], 1.0)
  return 1.0 / deg[:, 0:1]


def _comb_mm_body(eps_ref, y_ref, p0_ref, p1_ref, d0_ref, d1_ref, b_ref, w_ref,
                  o_ref):
  inv = _inv_deg(d0_ref[...], d1_ref[...])
  agg = (p0_ref[0] + p1_ref[0]) * inv
  h = (1.0 + eps_ref[0, 0]) * y_ref[...] + agg + b_ref[...]
  h = jnp.maximum(h, 0.0)
  o_ref[...] = jnp.dot(h, w_ref[...], preferred_element_type=jnp.float32)


def _comb_final_body(eps_ref, y_ref, p0_ref, p1_ref, d0_ref, d1_ref, b_ref,
                     o_ref):
  inv = _inv_deg(d0_ref[...], d1_ref[...])
  agg = (p0_ref[0] + p1_ref[0]) * inv
  o_ref[...] = (1.0 + eps_ref[0, 0]) * y_ref[...] + agg + b_ref[...]


_HALF = lambda half: (lambda i, h=half: (h, i, 0))
_DSPECS = [pl.BlockSpec((1, BLK, 16), _HALF(0)),
           pl.BlockSpec((1, BLK, 16), _HALF(1))]


def _combine_mm(eps_i, y, p, aggd, b, w):
  dy = y.shape[1]
  dout = w.shape[1]
  return pl.pallas_call(
      _comb_mm_body,
      grid=(N_PAD // BLK,),
      in_specs=[
          pl.BlockSpec(memory_space=pltpu.SMEM),
          pl.BlockSpec((BLK, dy), lambda i: (i, 0)),
          pl.BlockSpec((1, BLK, dy), _HALF(0)),
          pl.BlockSpec((1, BLK, dy), _HALF(1)),
          *_DSPECS,
          pl.BlockSpec((1, dy), lambda i: (0, 0)),
          pl.BlockSpec((dy, dout), lambda i: (0, 0)),
      ],
      out_specs=pl.BlockSpec((BLK, dout), lambda i: (i, 0)),
      out_shape=jax.ShapeDtypeStruct((N_PAD, dout), jnp.float32),
  )(eps_i.reshape(1, 1), y, p, p, aggd, aggd, b.reshape(1, dy), w)


def _combine_final(eps_i, y, p, aggd, b):
  dy = y.shape[1]
  return pl.pallas_call(
      _comb_final_body,
      grid=(N_PAD // BLK,),
      in_specs=[
          pl.BlockSpec(memory_space=pltpu.SMEM),
          pl.BlockSpec((BLK, dy), lambda i: (i, 0)),
          pl.BlockSpec((1, BLK, dy), _HALF(0)),
          pl.BlockSpec((1, BLK, dy), _HALF(1)),
          *_DSPECS,
          pl.BlockSpec((1, dy), lambda i: (0, 0)),
      ],
      out_specs=pl.BlockSpec((BLK, dy), lambda i: (i, 0)),
      out_shape=jax.ShapeDtypeStruct((N_PAD, dy), jnp.float32),
  )(eps_i.reshape(1, 1), y, p, p, aggd, aggd, b.reshape(1, dy))


def kernel(features, edge_index, W0, b0, W1, b1, W2, b2, W3, b3, eps):
  E = edge_index.shape[1]
  src, dst = edge_index[0], edge_index[1]
  if E % CH:
    padn = CH - E % CH
    junk = JUNK_ROW + (jnp.arange(padn, dtype=jnp.int32) % (N_PAD - N_NODES))
    src = jnp.concatenate([src, junk])
    dst = jnp.concatenate([dst, junk])
  src2 = src.reshape(-1, CH)
  dst2 = dst.reshape(-1, CH)
  nck = src2.shape[0]

  aggd = _sc_deg(nck)(dst2)
  y0 = _tc_matmul(features, W0)
  k128 = _sc_agg(128, nck)
  p0 = k128(y0, src2, dst2)
  y1 = _combine_mm(eps[0], y0, p0, aggd, b0, W1)
  p1 = k128(y1, src2, dst2)
  y2 = _combine_mm(eps[1], y1, p1, aggd, b1, W2)
  p2 = k128(y2, src2, dst2)
  W3p = jnp.pad(W3, ((0, 0), (0, 8)))
  y3 = _combine_mm(eps[2], y2, p2, aggd, b2, W3p)
  p3 = _sc_agg(48, nck)(y3, src2, dst2)
  out48 = _combine_final(eps[3], y3, p3, aggd, jnp.pad(b3, (0, 8)))
  return out48[:N_NODES, :40]


# speedup vs baseline: 1.0595x; 1.0595x over previous
import functools

import jax
import jax.numpy as jnp
from jax import lax
from jax.experimental import pallas as pl
from jax.experimental.pallas import tpu as pltpu
from jax.experimental.pallas import tpu_sc as plsc

N_NODES = 10000
N_PAD = 10240
JUNK_ROW = N_NODES
BLK = 512
CH = 128
NB = 2
SB = 20
N_TILES = 32
RPT = N_PAD // 16

_MESH = dict(core_axis_name="c", subcore_axis_name="s")
_SC_PARAMS = dict(
    compiler_params=pltpu.CompilerParams(use_tc_tiling_on_sc=False))


def _fill(ref, rows, width, vec):
  for r in range(rows):
    for q in range(width // 16):
      ref[r, pl.ds(q * 16, 16)] = vec


def _sc_deg(nch):
  @functools.partial(
      pl.kernel,
      mesh=plsc.VectorSubcoreMesh(**_MESH),
      out_type=jax.ShapeDtypeStruct((2, N_PAD, 16), jnp.float32),
      scratch_types=[
          pltpu.VMEM_SHARED((N_PAD, 16), jnp.float32),
          pltpu.VMEM((nch, CH), jnp.int32),
          pltpu.VMEM((CH, 16), jnp.float32),
          pltpu.VMEM((64, 16), jnp.float32),
          pltpu.SemaphoreType.DMA,
      ],
      **_SC_PARAMS,
  )
  def k(dst_hbm, out_hbm, acc, dst_v, ones_v, zblk, sem):
    c = lax.axis_index("c")
    s = lax.axis_index("s")
    wid = s * 2 + c
    _fill(ones_v, CH, 16, jnp.ones((16,), jnp.float32))
    _fill(zblk, 64, 16, jnp.zeros((16,), jnp.float32))
    row0 = s * RPT

    def zbody(j, carry):
      pltpu.sync_copy(zblk, acc.at[pl.ds(row0 + j * 64, 64)])
      return carry

    lax.fori_loop(0, RPT // 64, zbody, 0)
    pltpu.sync_copy(dst_hbm.at[pl.ds(wid * nch, nch)], dst_v)
    plsc.subcore_barrier()

    def group(g, carry):
      j0 = g * 4
      hs = [pltpu.async_copy(ones_v, acc.at[dst_v.at[j0 + b]], sem, add=True)
            for b in range(4)]
      for h in hs:
        h.wait()
      return carry

    lax.fori_loop(0, nch // 4, group, 0)
    plsc.subcore_barrier()
    pltpu.sync_copy(acc.at[pl.ds(row0, RPT)], out_hbm.at[c, pl.ds(row0, RPT)])

  return k


def _sc_agg(D, nch):
  @functools.partial(
      pl.kernel,
      mesh=plsc.VectorSubcoreMesh(**_MESH),
      out_type=jax.ShapeDtypeStruct((2, N_PAD, D), jnp.float32),
      scratch_types=[
          pltpu.VMEM_SHARED((N_PAD, D), jnp.float32),
          pltpu.VMEM((SB, CH), jnp.int32),
          pltpu.VMEM((SB, CH), jnp.int32),
      ] + [pltpu.VMEM((CH, D), jnp.float32) for _ in range(NB)] + [
          pltpu.VMEM((16, D), jnp.float32),
          pltpu.SemaphoreType.DMA,
          pltpu.SemaphoreType.DMA,
      ],
      **_SC_PARAMS,
  )
  def k(y_hbm, src_hbm, dst_hbm, out_hbm, acc, src_v, dst_v,
        *bufs_zblk_sems):
    rows = list(bufs_zblk_sems[:NB])
    zblk, gsem, ssem = bufs_zblk_sems[NB:]
    c = lax.axis_index("c")
    s = lax.axis_index("s")
    wid = s * 2 + c
    _fill(zblk, 16, D, jnp.zeros((16,), jnp.float32))
    row0 = s * RPT

    def zbody(j, carry):
      pltpu.sync_copy(zblk, acc.at[pl.ds(row0 + j * 16, 16)])
      return carry

    lax.fori_loop(0, RPT // 16, zbody, 0)
    plsc.subcore_barrier()

    def sblock(sb, carry):
      chunk0 = wid * nch + sb * SB
      pltpu.sync_copy(src_hbm.at[pl.ds(chunk0, SB)], src_v)
      pltpu.sync_copy(dst_hbm.at[pl.ds(chunk0, SB)], dst_v)

      def group(g, carry2):
        j0 = g * NB
        ghs = [pltpu.async_copy(y_hbm.at[src_v.at[j0 + b]], rows[b], gsem)
               for b in range(NB)]
        shs = []
        for b in range(NB):
          ghs[b].wait()
          shs.append(pltpu.async_copy(rows[b], acc.at[dst_v.at[j0 + b]], ssem,
                                      add=True))
        for h in shs:
          h.wait()
        return carry2

      lax.fori_loop(0, SB // NB, group, 0)
      return carry

    lax.fori_loop(0, nch // SB, sblock, 0)
    plsc.subcore_barrier()
    pltpu.sync_copy(acc.at[pl.ds(row0, RPT)], out_hbm.at[c, pl.ds(row0, RPT)])

  return k


def _mm_body(x_ref, w_ref, o_ref):
  o_ref[...] = jnp.dot(x_ref[...], w_ref[...], preferred_element_type=jnp.float32)


def _tc_matmul(x, w):
  kdim = x.shape[1]
  dout = w.shape[1]
  return pl.pallas_call(
      _mm_body,
      grid=(N_PAD // BLK,),
      in_specs=[
          pl.BlockSpec((BLK, kdim), lambda i: (i, 0)),
          pl.BlockSpec((kdim, dout), lambda i: (0, 0)),
      ],
      out_specs=pl.BlockSpec((BLK, dout), lambda i: (i, 0)),
      out_shape=jax.ShapeDtypeStruct((N_PAD, dout), jnp.float32),
  )(x, w)


def _inv_deg(d0, d1):
  deg = jnp.maximum(d0[0] + d1[0], 1.0)
  return 1.0 / deg[:, 0:1]


def _comb_mm_body(eps_ref, y_ref, p0_ref, p1_ref, d0_ref, d1_ref, b_ref, w_ref,
                  o_ref):
  inv = _inv_deg(d0_ref[...], d1_ref[...])
  agg = (p0_ref[0] + p1_ref[0]) * inv
  h = (1.0 + eps_ref[0, 0]) * y_ref[...] + agg + b_ref[...]
  h = jnp.maximum(h, 0.0)
  o_ref[...] = jnp.dot(h, w_ref[...], preferred_element_type=jnp.float32)


def _comb_final_body(eps_ref, y_ref, p0_ref, p1_ref, d0_ref, d1_ref, b_ref,
                     o_ref):
  inv = _inv_deg(d0_ref[...], d1_ref[...])
  agg = (p0_ref[0] + p1_ref[0]) * inv
  o_ref[...] = (1.0 + eps_ref[0, 0]) * y_ref[...] + agg + b_ref[...]


_HALF = lambda half: (lambda i, h=half: (h, i, 0))
_DSPECS = [pl.BlockSpec((1, BLK, 16), _HALF(0)),
           pl.BlockSpec((1, BLK, 16), _HALF(1))]


def _combine_mm(eps_i, y, p, aggd, b, w):
  dy = y.shape[1]
  dout = w.shape[1]
  return pl.pallas_call(
      _comb_mm_body,
      grid=(N_PAD // BLK,),
      in_specs=[
          pl.BlockSpec(memory_space=pltpu.SMEM),
          pl.BlockSpec((BLK, dy), lambda i: (i, 0)),
          pl.BlockSpec((1, BLK, dy), _HALF(0)),
          pl.BlockSpec((1, BLK, dy), _HALF(1)),
          *_DSPECS,
          pl.BlockSpec((1, dy), lambda i: (0, 0)),
          pl.BlockSpec((dy, dout), lambda i: (0, 0)),
      ],
      out_specs=pl.BlockSpec((BLK, dout), lambda i: (i, 0)),
      out_shape=jax.ShapeDtypeStruct((N_PAD, dout), jnp.float32),
  )(eps_i.reshape(1, 1), y, p, p, aggd, aggd, b.reshape(1, dy), w)


def _combine_final(eps_i, y, p, aggd, b):
  dy = y.shape[1]
  return pl.pallas_call(
      _comb_final_body,
      grid=(N_PAD // BLK,),
      in_specs=[
          pl.BlockSpec(memory_space=pltpu.SMEM),
          pl.BlockSpec((BLK, dy), lambda i: (i, 0)),
          pl.BlockSpec((1, BLK, dy), _HALF(0)),
          pl.BlockSpec((1, BLK, dy), _HALF(1)),
          *_DSPECS,
          pl.BlockSpec((1, dy), lambda i: (0, 0)),
      ],
      out_specs=pl.BlockSpec((BLK, dy), lambda i: (i, 0)),
      out_shape=jax.ShapeDtypeStruct((N_PAD, dy), jnp.float32),
  )(eps_i.reshape(1, 1), y, p, p, aggd, aggd, b.reshape(1, dy))


def kernel(features, edge_index, W0, b0, W1, b1, W2, b2, W3, b3, eps):
  E = edge_index.shape[1]
  src, dst = edge_index[0], edge_index[1]
  gran = N_TILES * CH * SB
  e_pad = ((E + gran - 1) // gran) * gran
  padn = e_pad - E
  if padn:
    junk = JUNK_ROW + (jnp.arange(padn, dtype=jnp.int32) % (N_PAD - N_NODES))
    src = jnp.concatenate([src, junk])
    dst = jnp.concatenate([dst, junk])
  src2 = src.reshape(-1, CH)
  dst2 = dst.reshape(-1, CH)
  nch = e_pad // (N_TILES * CH)

  aggd = _sc_deg(nch)(dst2)
  y0 = _tc_matmul(features, W0)
  p0 = _sc_agg(128, nch)(y0, src2, dst2)
  y1 = _combine_mm(eps[0], y0, p0, aggd, b0, W1)
  p1 = _sc_agg(128, nch)(y1, src2, dst2)
  y2 = _combine_mm(eps[1], y1, p1, aggd, b1, W2)
  p2 = _sc_agg(128, nch)(y2, src2, dst2)
  W3p = jnp.pad(W3, ((0, 0), (0, 8)))
  y3 = _combine_mm(eps[2], y2, p2, aggd, b2, W3p)
  p3 = _sc_agg(48, nch)(y3, src2, dst2)
  out48 = _combine_final(eps[3], y3, p3, aggd, jnp.pad(b3, (0, 8)))
  return out48[:N_NODES, :40]


# speedup vs baseline: 1.0783x; 1.0178x over previous
import functools

import jax
import jax.numpy as jnp
from jax import lax
from jax.experimental import pallas as pl
from jax.experimental.pallas import tpu as pltpu
from jax.experimental.pallas import tpu_sc as plsc

N_NODES = 10000
N_PAD = 10240
JUNK_ROW = N_NODES
BLK = 512
CH = 128
NB = 2
SB = 40
N_TILES = 32
RPT = N_PAD // 16

_MESH = dict(core_axis_name="c", subcore_axis_name="s")
_SC_PARAMS = dict(
    compiler_params=pltpu.CompilerParams(use_tc_tiling_on_sc=False))


def _fill(ref, rows, width, vec):
  for r in range(rows):
    for q in range(width // 16):
      ref[r, pl.ds(q * 16, 16)] = vec


def _sc_deg(nch):
  @functools.partial(
      pl.kernel,
      mesh=plsc.VectorSubcoreMesh(**_MESH),
      out_type=jax.ShapeDtypeStruct((2, N_PAD, 16), jnp.float32),
      scratch_types=[
          pltpu.VMEM_SHARED((N_PAD, 16), jnp.float32),
          pltpu.VMEM((nch, CH), jnp.int32),
          pltpu.VMEM((CH, 16), jnp.float32),
          pltpu.VMEM((64, 16), jnp.float32),
          pltpu.SemaphoreType.DMA,
      ],
      **_SC_PARAMS,
  )
  def k(dst_hbm, out_hbm, acc, dst_v, ones_v, zblk, sem):
    c = lax.axis_index("c")
    s = lax.axis_index("s")
    wid = s * 2 + c
    _fill(ones_v, CH, 16, jnp.ones((16,), jnp.float32))
    _fill(zblk, 64, 16, jnp.zeros((16,), jnp.float32))
    row0 = s * RPT

    def zbody(j, carry):
      pltpu.sync_copy(zblk, acc.at[pl.ds(row0 + j * 64, 64)])
      return carry

    lax.fori_loop(0, RPT // 64, zbody, 0)
    pltpu.sync_copy(dst_hbm.at[pl.ds(wid * nch, nch)], dst_v)
    plsc.subcore_barrier()

    def group(g, carry):
      j0 = g * 4
      hs = [pltpu.async_copy(ones_v, acc.at[dst_v.at[j0 + b]], sem, add=True)
            for b in range(4)]
      for h in hs:
        h.wait()
      return carry

    lax.fori_loop(0, nch // 4, group, 0)
    plsc.subcore_barrier()
    pltpu.sync_copy(acc.at[pl.ds(row0, RPT)], out_hbm.at[c, pl.ds(row0, RPT)])

  return k


def _sc_agg(D, nch):
  @functools.partial(
      pl.kernel,
      mesh=plsc.VectorSubcoreMesh(**_MESH),
      out_type=jax.ShapeDtypeStruct((2, N_PAD, D), jnp.float32),
      scratch_types=[
          pltpu.VMEM_SHARED((N_PAD, D), jnp.float32),
          pltpu.VMEM((SB, CH), jnp.int32),
          pltpu.VMEM((SB, CH), jnp.int32),
      ] + [pltpu.VMEM((CH, D), jnp.float32) for _ in range(NB)] + [
          pltpu.VMEM((16, D), jnp.float32),
          pltpu.SemaphoreType.DMA,
          pltpu.SemaphoreType.DMA,
      ],
      **_SC_PARAMS,
  )
  def k(y_hbm, src_hbm, dst_hbm, out_hbm, acc, src_v, dst_v,
        *bufs_zblk_sems):
    rows = list(bufs_zblk_sems[:NB])
    zblk, gsem, ssem = bufs_zblk_sems[NB:]
    c = lax.axis_index("c")
    s = lax.axis_index("s")
    wid = s * 2 + c
    _fill(zblk, 16, D, jnp.zeros((16,), jnp.float32))
    row0 = s * RPT

    def zbody(j, carry):
      pltpu.sync_copy(zblk, acc.at[pl.ds(row0 + j * 16, 16)])
      return carry

    lax.fori_loop(0, RPT // 16, zbody, 0)
    plsc.subcore_barrier()

    def sblock(sb, carry):
      chunk0 = wid * nch + sb * SB
      pltpu.sync_copy(src_hbm.at[pl.ds(chunk0, SB)], src_v)
      pltpu.sync_copy(dst_hbm.at[pl.ds(chunk0, SB)], dst_v)

      def group(g, carry2):
        j0 = g * NB
        ghs = [pltpu.async_copy(y_hbm.at[src_v.at[j0 + b]], rows[b], gsem)
               for b in range(NB)]
        shs = []
        for b in range(NB):
          ghs[b].wait()
          shs.append(pltpu.async_copy(rows[b], acc.at[dst_v.at[j0 + b]], ssem,
                                      add=True))
        for h in shs:
          h.wait()
        return carry2

      lax.fori_loop(0, SB // NB, group, 0)
      return carry

    lax.fori_loop(0, nch // SB, sblock, 0)
    plsc.subcore_barrier()
    pltpu.sync_copy(acc.at[pl.ds(row0, RPT)], out_hbm.at[c, pl.ds(row0, RPT)])

  return k


def _mm_body(x_ref, w_ref, o_ref):
  o_ref[...] = jnp.dot(x_ref[...], w_ref[...], preferred_element_type=jnp.float32)


def _tc_matmul(x, w):
  kdim = x.shape[1]
  dout = w.shape[1]
  return pl.pallas_call(
      _mm_body,
      grid=(N_PAD // BLK,),
      in_specs=[
          pl.BlockSpec((BLK, kdim), lambda i: (i, 0)),
          pl.BlockSpec((kdim, dout), lambda i: (0, 0)),
      ],
      out_specs=pl.BlockSpec((BLK, dout), lambda i: (i, 0)),
      out_shape=jax.ShapeDtypeStruct((N_PAD, dout), jnp.float32),
  )(x, w)


def _inv_deg(d0, d1):
  deg = jnp.maximum(d0[0] + d1[0], 1.0)
  return 1.0 / deg[:, 0:1]


def _comb_mm_body(eps_ref, y_ref, p0_ref, p1_ref, d0_ref, d1_ref, b_ref, w_ref,
                  o_ref):
  inv = _inv_deg(d0_ref[...], d1_ref[...])
  agg = (p0_ref[0] + p1_ref[0]) * inv
  h = (1.0 + eps_ref[0, 0]) * y_ref[...] + agg + b_ref[...]
  h = jnp.maximum(h, 0.0)
  o_ref[...] = jnp.dot(h, w_ref[...], preferred_element_type=jnp.float32)


def _comb_final_body(eps_ref, y_ref, p0_ref, p1_ref, d0_ref, d1_ref, b_ref,
                     o_ref):
  inv = _inv_deg(d0_ref[...], d1_ref[...])
  agg = (p0_ref[0] + p1_ref[0]) * inv
  o_ref[...] = (1.0 + eps_ref[0, 0]) * y_ref[...] + agg + b_ref[...]


_HALF = lambda half: (lambda i, h=half: (h, i, 0))
_DSPECS = [pl.BlockSpec((1, BLK, 16), _HALF(0)),
           pl.BlockSpec((1, BLK, 16), _HALF(1))]


def _combine_mm(eps_i, y, p, aggd, b, w):
  dy = y.shape[1]
  dout = w.shape[1]
  return pl.pallas_call(
      _comb_mm_body,
      grid=(N_PAD // BLK,),
      in_specs=[
          pl.BlockSpec(memory_space=pltpu.SMEM),
          pl.BlockSpec((BLK, dy), lambda i: (i, 0)),
          pl.BlockSpec((1, BLK, dy), _HALF(0)),
          pl.BlockSpec((1, BLK, dy), _HALF(1)),
          *_DSPECS,
          pl.BlockSpec((1, dy), lambda i: (0, 0)),
          pl.BlockSpec((dy, dout), lambda i: (0, 0)),
      ],
      out_specs=pl.BlockSpec((BLK, dout), lambda i: (i, 0)),
      out_shape=jax.ShapeDtypeStruct((N_PAD, dout), jnp.float32),
  )(eps_i.reshape(1, 1), y, p, p, aggd, aggd, b.reshape(1, dy), w)


def _combine_final(eps_i, y, p, aggd, b):
  dy = y.shape[1]
  return pl.pallas_call(
      _comb_final_body,
      grid=(N_PAD // BLK,),
      in_specs=[
          pl.BlockSpec(memory_space=pltpu.SMEM),
          pl.BlockSpec((BLK, dy), lambda i: (i, 0)),
          pl.BlockSpec((1, BLK, dy), _HALF(0)),
          pl.BlockSpec((1, BLK, dy), _HALF(1)),
          *_DSPECS,
          pl.BlockSpec((1, dy), lambda i: (0, 0)),
      ],
      out_specs=pl.BlockSpec((BLK, dy), lambda i: (i, 0)),
      out_shape=jax.ShapeDtypeStruct((N_PAD, dy), jnp.float32),
  )(eps_i.reshape(1, 1), y, p, p, aggd, aggd, b.reshape(1, dy))


def kernel(features, edge_index, W0, b0, W1, b1, W2, b2, W3, b3, eps):
  E = edge_index.shape[1]
  src, dst = edge_index[0], edge_index[1]
  gran = N_TILES * CH * SB
  e_pad = ((E + gran - 1) // gran) * gran
  padn = e_pad - E
  if padn:
    junk = JUNK_ROW + (jnp.arange(padn, dtype=jnp.int32) % (N_PAD - N_NODES))
    src = jnp.concatenate([src, junk])
    dst = jnp.concatenate([dst, junk])
  src2 = src.reshape(-1, CH)
  dst2 = dst.reshape(-1, CH)
  nch = e_pad // (N_TILES * CH)

  aggd = _sc_deg(nch)(dst2)
  y0 = _tc_matmul(features, W0)
  p0 = _sc_agg(128, nch)(y0, src2, dst2)
  y1 = _combine_mm(eps[0], y0, p0, aggd, b0, W1)
  p1 = _sc_agg(128, nch)(y1, src2, dst2)
  y2 = _combine_mm(eps[1], y1, p1, aggd, b1, W2)
  p2 = _sc_agg(128, nch)(y2, src2, dst2)
  W3p = jnp.pad(W3, ((0, 0), (0, 8)))
  y3 = _combine_mm(eps[2], y2, p2, aggd, b2, W3p)
  p3 = _sc_agg(48, nch)(y3, src2, dst2)
  out48 = _combine_final(eps[3], y3, p3, aggd, jnp.pad(b3, (0, 8)))
  return out48[:N_NODES, :40]


# speedup vs baseline: 1.1203x; 1.0390x over previous
import functools

import jax
import jax.numpy as jnp
from jax import lax
from jax.experimental import pallas as pl
from jax.experimental.pallas import tpu as pltpu
from jax.experimental.pallas import tpu_sc as plsc

N_NODES = 10000
N_PAD = 10240
JUNK_ROW = N_NODES
BLK = 1024
CH = 128
NB = 2
SB = 40
N_TILES = 32
RPT = N_PAD // 16

_MESH = dict(core_axis_name="c", subcore_axis_name="s")
_SC_PARAMS = dict(
    compiler_params=pltpu.CompilerParams(use_tc_tiling_on_sc=False))


def _fill(ref, rows, width, vec):
  for r in range(rows):
    for q in range(width // 16):
      ref[r, pl.ds(q * 16, 16)] = vec


def _sc_deg(nch):
  @functools.partial(
      pl.kernel,
      mesh=plsc.VectorSubcoreMesh(**_MESH),
      out_type=jax.ShapeDtypeStruct((2, N_PAD, 16), jnp.float32),
      scratch_types=[
          pltpu.VMEM_SHARED((N_PAD, 16), jnp.float32),
          pltpu.VMEM((nch, CH), jnp.int32),
          pltpu.VMEM((CH, 16), jnp.float32),
          pltpu.VMEM((64, 16), jnp.float32),
          pltpu.SemaphoreType.DMA,
      ],
      **_SC_PARAMS,
  )
  def k(dst_hbm, out_hbm, acc, dst_v, ones_v, zblk, sem):
    c = lax.axis_index("c")
    s = lax.axis_index("s")
    wid = s * 2 + c
    _fill(ones_v, CH, 16, jnp.ones((16,), jnp.float32))
    _fill(zblk, 64, 16, jnp.zeros((16,), jnp.float32))
    row0 = s * RPT

    def zbody(j, carry):
      pltpu.sync_copy(zblk, acc.at[pl.ds(row0 + j * 64, 64)])
      return carry

    lax.fori_loop(0, RPT // 64, zbody, 0)
    pltpu.sync_copy(dst_hbm.at[pl.ds(wid * nch, nch)], dst_v)
    plsc.subcore_barrier()

    def group(g, carry):
      j0 = g * 4
      hs = [pltpu.async_copy(ones_v, acc.at[dst_v.at[j0 + b]], sem, add=True)
            for b in range(4)]
      for h in hs:
        h.wait()
      return carry

    lax.fori_loop(0, nch // 4, group, 0)
    plsc.subcore_barrier()
    pltpu.sync_copy(acc.at[pl.ds(row0, RPT)], out_hbm.at[c, pl.ds(row0, RPT)])

  return k


def _sc_agg(D, nch):
  @functools.partial(
      pl.kernel,
      mesh=plsc.VectorSubcoreMesh(**_MESH),
      out_type=jax.ShapeDtypeStruct((2, N_PAD, D), jnp.float32),
      scratch_types=[
          pltpu.VMEM_SHARED((N_PAD, D), jnp.float32),
          pltpu.VMEM((SB, CH), jnp.int32),
          pltpu.VMEM((SB, CH), jnp.int32),
      ] + [pltpu.VMEM((CH, D), jnp.float32) for _ in range(NB)] + [
          pltpu.VMEM((16, D), jnp.float32),
          pltpu.SemaphoreType.DMA,
          pltpu.SemaphoreType.DMA,
      ],
      **_SC_PARAMS,
  )
  def k(y_hbm, src_hbm, dst_hbm, out_hbm, acc, src_v, dst_v,
        *bufs_zblk_sems):
    rows = list(bufs_zblk_sems[:NB])
    zblk, gsem, ssem = bufs_zblk_sems[NB:]
    c = lax.axis_index("c")
    s = lax.axis_index("s")
    wid = s * 2 + c
    _fill(zblk, 16, D, jnp.zeros((16,), jnp.float32))
    row0 = s * RPT

    def zbody(j, carry):
      pltpu.sync_copy(zblk, acc.at[pl.ds(row0 + j * 16, 16)])
      return carry

    lax.fori_loop(0, RPT // 16, zbody, 0)
    plsc.subcore_barrier()

    def sblock(sb, carry):
      chunk0 = wid * nch + sb * SB
      pltpu.sync_copy(src_hbm.at[pl.ds(chunk0, SB)], src_v)
      pltpu.sync_copy(dst_hbm.at[pl.ds(chunk0, SB)], dst_v)

      def group(g, carry2):
        j0 = g * NB
        ghs = [pltpu.async_copy(y_hbm.at[src_v.at[j0 + b]], rows[b], gsem)
               for b in range(NB)]
        shs = []
        for b in range(NB):
          ghs[b].wait()
          shs.append(pltpu.async_copy(rows[b], acc.at[dst_v.at[j0 + b]], ssem,
                                      add=True))
        for h in shs:
          h.wait()
        return carry2

      lax.fori_loop(0, SB // NB, group, 0)
      return carry

    lax.fori_loop(0, nch // SB, sblock, 0)
    plsc.subcore_barrier()
    pltpu.sync_copy(acc.at[pl.ds(row0, RPT)], out_hbm.at[c, pl.ds(row0, RPT)])

  return k


def _mm_body(x_ref, w_ref, o_ref):
  o_ref[...] = jnp.dot(x_ref[...], w_ref[...], preferred_element_type=jnp.float32)


def _tc_matmul(x, w):
  kdim = x.shape[1]
  dout = w.shape[1]
  return pl.pallas_call(
      _mm_body,
      grid=(N_PAD // BLK,),
      in_specs=[
          pl.BlockSpec((BLK, kdim), lambda i: (i, 0)),
          pl.BlockSpec((kdim, dout), lambda i: (0, 0)),
      ],
      out_specs=pl.BlockSpec((BLK, dout), lambda i: (i, 0)),
      out_shape=jax.ShapeDtypeStruct((N_PAD, dout), jnp.float32),
  )(x, w)


def _inv_deg(d0, d1):
  deg = jnp.maximum(d0[0] + d1[0], 1.0)
  return 1.0 / deg[:, 0:1]


def _comb_mm_body(eps_ref, y_ref, p0_ref, p1_ref, d0_ref, d1_ref, b_ref, w_ref,
                  o_ref):
  inv = _inv_deg(d0_ref[...], d1_ref[...])
  agg = (p0_ref[0] + p1_ref[0]) * inv
  h = (1.0 + eps_ref[0, 0]) * y_ref[...] + agg + b_ref[...]
  h = jnp.maximum(h, 0.0)
  o_ref[...] = jnp.dot(h, w_ref[...], preferred_element_type=jnp.float32)


def _comb_final_body(eps_ref, y_ref, p0_ref, p1_ref, d0_ref, d1_ref, b_ref,
                     o_ref):
  inv = _inv_deg(d0_ref[...], d1_ref[...])
  agg = (p0_ref[0] + p1_ref[0]) * inv
  o_ref[...] = (1.0 + eps_ref[0, 0]) * y_ref[...] + agg + b_ref[...]


_HALF = lambda half: (lambda i, h=half: (h, i, 0))
_DSPECS = [pl.BlockSpec((1, BLK, 16), _HALF(0)),
           pl.BlockSpec((1, BLK, 16), _HALF(1))]


def _combine_mm(eps_i, y, p, aggd, b, w):
  dy = y.shape[1]
  dout = w.shape[1]
  return pl.pallas_call(
      _comb_mm_body,
      grid=(N_PAD // BLK,),
      in_specs=[
          pl.BlockSpec(memory_space=pltpu.SMEM),
          pl.BlockSpec((BLK, dy), lambda i: (i, 0)),
          pl.BlockSpec((1, BLK, dy), _HALF(0)),
          pl.BlockSpec((1, BLK, dy), _HALF(1)),
          *_DSPECS,
          pl.BlockSpec((1, dy), lambda i: (0, 0)),
          pl.BlockSpec((dy, dout), lambda i: (0, 0)),
      ],
      out_specs=pl.BlockSpec((BLK, dout), lambda i: (i, 0)),
      out_shape=jax.ShapeDtypeStruct((N_PAD, dout), jnp.float32),
  )(eps_i.reshape(1, 1), y, p, p, aggd, aggd, b.reshape(1, dy), w)


def _combine_final(eps_i, y, p, aggd, b):
  dy = y.shape[1]
  return pl.pallas_call(
      _comb_final_body,
      grid=(N_PAD // BLK,),
      in_specs=[
          pl.BlockSpec(memory_space=pltpu.SMEM),
          pl.BlockSpec((BLK, dy), lambda i: (i, 0)),
          pl.BlockSpec((1, BLK, dy), _HALF(0)),
          pl.BlockSpec((1, BLK, dy), _HALF(1)),
          *_DSPECS,
          pl.BlockSpec((1, dy), lambda i: (0, 0)),
      ],
      out_specs=pl.BlockSpec((BLK, dy), lambda i: (i, 0)),
      out_shape=jax.ShapeDtypeStruct((N_PAD, dy), jnp.float32),
  )(eps_i.reshape(1, 1), y, p, p, aggd, aggd, b.reshape(1, dy))


def kernel(features, edge_index, W0, b0, W1, b1, W2, b2, W3, b3, eps):
  E = edge_index.shape[1]
  src, dst = edge_index[0], edge_index[1]
  gran = N_TILES * CH * SB
  e_pad = ((E + gran - 1) // gran) * gran
  padn = e_pad - E
  if padn:
    junk = JUNK_ROW + (jnp.arange(padn, dtype=jnp.int32) % (N_PAD - N_NODES))
    src = jnp.concatenate([src, junk])
    dst = jnp.concatenate([dst, junk])
  src2 = src.reshape(-1, CH)
  dst2 = dst.reshape(-1, CH)
  nch = e_pad // (N_TILES * CH)

  aggd = _sc_deg(nch)(dst2)
  y0 = _tc_matmul(features, W0)
  p0 = _sc_agg(128, nch)(y0, src2, dst2)
  y1 = _combine_mm(eps[0], y0, p0, aggd, b0, W1)
  p1 = _sc_agg(128, nch)(y1, src2, dst2)
  y2 = _combine_mm(eps[1], y1, p1, aggd, b1, W2)
  p2 = _sc_agg(128, nch)(y2, src2, dst2)
  W3p = jnp.pad(W3, ((0, 0), (0, 8)))
  y3 = _combine_mm(eps[2], y2, p2, aggd, b2, W3p)
  p3 = _sc_agg(48, nch)(y3, src2, dst2)
  out48 = _combine_final(eps[3], y3, p3, aggd, jnp.pad(b3, (0, 8)))
  return out48[:N_NODES, :40]


# speedup vs baseline: 1.1296x; 1.0083x over previous
import functools

import jax
import jax.numpy as jnp
from jax import lax
from jax.experimental import pallas as pl
from jax.experimental.pallas import tpu as pltpu
from jax.experimental.pallas import tpu_sc as plsc

N_NODES = 10000
N_PAD = 10240
JUNK_ROW = N_NODES
BLK = 2048
CH = 128
NB = 2
SB = 40
N_TILES = 32
RPT = N_PAD // 16

_MESH = dict(core_axis_name="c", subcore_axis_name="s")
_SC_PARAMS = dict(
    compiler_params=pltpu.CompilerParams(use_tc_tiling_on_sc=False))


def _fill(ref, rows, width, vec):
  for r in range(rows):
    for q in range(width // 16):
      ref[r, pl.ds(q * 16, 16)] = vec


def _sc_deg(nch):
  @functools.partial(
      pl.kernel,
      mesh=plsc.VectorSubcoreMesh(**_MESH),
      out_type=jax.ShapeDtypeStruct((2, N_PAD, 16), jnp.float32),
      scratch_types=[
          pltpu.VMEM_SHARED((N_PAD, 16), jnp.float32),
          pltpu.VMEM((nch, CH), jnp.int32),
          pltpu.VMEM((CH, 16), jnp.float32),
          pltpu.VMEM((64, 16), jnp.float32),
          pltpu.SemaphoreType.DMA,
      ],
      **_SC_PARAMS,
  )
  def k(dst_hbm, out_hbm, acc, dst_v, ones_v, zblk, sem):
    c = lax.axis_index("c")
    s = lax.axis_index("s")
    wid = s * 2 + c
    _fill(ones_v, CH, 16, jnp.ones((16,), jnp.float32))
    _fill(zblk, 64, 16, jnp.zeros((16,), jnp.float32))
    row0 = s * RPT

    def zbody(j, carry):
      pltpu.sync_copy(zblk, acc.at[pl.ds(row0 + j * 64, 64)])
      return carry

    lax.fori_loop(0, RPT // 64, zbody, 0)
    pltpu.sync_copy(dst_hbm.at[pl.ds(wid * nch, nch)], dst_v)
    plsc.subcore_barrier()

    def group(g, carry):
      j0 = g * 4
      hs = [pltpu.async_copy(ones_v, acc.at[dst_v.at[j0 + b]], sem, add=True)
            for b in range(4)]
      for h in hs:
        h.wait()
      return carry

    lax.fori_loop(0, nch // 4, group, 0)
    plsc.subcore_barrier()
    pltpu.sync_copy(acc.at[pl.ds(row0, RPT)], out_hbm.at[c, pl.ds(row0, RPT)])

  return k


def _sc_agg(D, nch):
  @functools.partial(
      pl.kernel,
      mesh=plsc.VectorSubcoreMesh(**_MESH),
      out_type=jax.ShapeDtypeStruct((2, N_PAD, D), jnp.float32),
      scratch_types=[
          pltpu.VMEM_SHARED((N_PAD, D), jnp.float32),
          pltpu.VMEM((SB, CH), jnp.int32),
          pltpu.VMEM((SB, CH), jnp.int32),
      ] + [pltpu.VMEM((CH, D), jnp.float32) for _ in range(NB)] + [
          pltpu.VMEM((16, D), jnp.float32),
          pltpu.SemaphoreType.DMA,
          pltpu.SemaphoreType.DMA,
      ],
      **_SC_PARAMS,
  )
  def k(y_hbm, src_hbm, dst_hbm, out_hbm, acc, src_v, dst_v,
        *bufs_zblk_sems):
    rows = list(bufs_zblk_sems[:NB])
    zblk, gsem, ssem = bufs_zblk_sems[NB:]
    c = lax.axis_index("c")
    s = lax.axis_index("s")
    wid = s * 2 + c
    _fill(zblk, 16, D, jnp.zeros((16,), jnp.float32))
    row0 = s * RPT

    def zbody(j, carry):
      pltpu.sync_copy(zblk, acc.at[pl.ds(row0 + j * 16, 16)])
      return carry

    lax.fori_loop(0, RPT // 16, zbody, 0)
    plsc.subcore_barrier()

    def sblock(sb, carry):
      chunk0 = wid * nch + sb * SB
      pltpu.sync_copy(src_hbm.at[pl.ds(chunk0, SB)], src_v)
      pltpu.sync_copy(dst_hbm.at[pl.ds(chunk0, SB)], dst_v)

      def group(g, carry2):
        j0 = g * NB
        ghs = [pltpu.async_copy(y_hbm.at[src_v.at[j0 + b]], rows[b], gsem)
               for b in range(NB)]
        shs = []
        for b in range(NB):
          ghs[b].wait()
          shs.append(pltpu.async_copy(rows[b], acc.at[dst_v.at[j0 + b]], ssem,
                                      add=True))
        for h in shs:
          h.wait()
        return carry2

      lax.fori_loop(0, SB // NB, group, 0)
      return carry

    lax.fori_loop(0, nch // SB, sblock, 0)
    plsc.subcore_barrier()
    pltpu.sync_copy(acc.at[pl.ds(row0, RPT)], out_hbm.at[c, pl.ds(row0, RPT)])

  return k


def _mm_body(x_ref, w_ref, o_ref):
  o_ref[...] = jnp.dot(x_ref[...], w_ref[...], preferred_element_type=jnp.float32)


def _tc_matmul(x, w):
  kdim = x.shape[1]
  dout = w.shape[1]
  return pl.pallas_call(
      _mm_body,
      grid=(N_PAD // BLK,),
      in_specs=[
          pl.BlockSpec((BLK, kdim), lambda i: (i, 0)),
          pl.BlockSpec((kdim, dout), lambda i: (0, 0)),
      ],
      out_specs=pl.BlockSpec((BLK, dout), lambda i: (i, 0)),
      out_shape=jax.ShapeDtypeStruct((N_PAD, dout), jnp.float32),
  )(x, w)


def _inv_deg(d0, d1):
  deg = jnp.maximum(d0[0] + d1[0], 1.0)
  return 1.0 / deg[:, 0:1]


def _comb_mm_body(eps_ref, y_ref, p0_ref, p1_ref, d0_ref, d1_ref, b_ref, w_ref,
                  o_ref):
  inv = _inv_deg(d0_ref[...], d1_ref[...])
  agg = (p0_ref[0] + p1_ref[0]) * inv
  h = (1.0 + eps_ref[0, 0]) * y_ref[...] + agg + b_ref[...]
  h = jnp.maximum(h, 0.0)
  o_ref[...] = jnp.dot(h, w_ref[...], preferred_element_type=jnp.float32)


def _comb_final_body(eps_ref, y_ref, p0_ref, p1_ref, d0_ref, d1_ref, b_ref,
                     o_ref):
  inv = _inv_deg(d0_ref[...], d1_ref[...])
  agg = (p0_ref[0] + p1_ref[0]) * inv
  o_ref[...] = (1.0 + eps_ref[0, 0]) * y_ref[...] + agg + b_ref[...]


_HALF = lambda half: (lambda i, h=half: (h, i, 0))
_DSPECS = [pl.BlockSpec((1, BLK, 16), _HALF(0)),
           pl.BlockSpec((1, BLK, 16), _HALF(1))]


def _combine_mm(eps_i, y, p, aggd, b, w):
  dy = y.shape[1]
  dout = w.shape[1]
  return pl.pallas_call(
      _comb_mm_body,
      grid=(N_PAD // BLK,),
      in_specs=[
          pl.BlockSpec(memory_space=pltpu.SMEM),
          pl.BlockSpec((BLK, dy), lambda i: (i, 0)),
          pl.BlockSpec((1, BLK, dy), _HALF(0)),
          pl.BlockSpec((1, BLK, dy), _HALF(1)),
          *_DSPECS,
          pl.BlockSpec((1, dy), lambda i: (0, 0)),
          pl.BlockSpec((dy, dout), lambda i: (0, 0)),
      ],
      out_specs=pl.BlockSpec((BLK, dout), lambda i: (i, 0)),
      out_shape=jax.ShapeDtypeStruct((N_PAD, dout), jnp.float32),
  )(eps_i.reshape(1, 1), y, p, p, aggd, aggd, b.reshape(1, dy), w)


def _combine_final(eps_i, y, p, aggd, b):
  dy = y.shape[1]
  return pl.pallas_call(
      _comb_final_body,
      grid=(N_PAD // BLK,),
      in_specs=[
          pl.BlockSpec(memory_space=pltpu.SMEM),
          pl.BlockSpec((BLK, dy), lambda i: (i, 0)),
          pl.BlockSpec((1, BLK, dy), _HALF(0)),
          pl.BlockSpec((1, BLK, dy), _HALF(1)),
          *_DSPECS,
          pl.BlockSpec((1, dy), lambda i: (0, 0)),
      ],
      out_specs=pl.BlockSpec((BLK, dy), lambda i: (i, 0)),
      out_shape=jax.ShapeDtypeStruct((N_PAD, dy), jnp.float32),
  )(eps_i.reshape(1, 1), y, p, p, aggd, aggd, b.reshape(1, dy))


def kernel(features, edge_index, W0, b0, W1, b1, W2, b2, W3, b3, eps):
  E = edge_index.shape[1]
  src, dst = edge_index[0], edge_index[1]
  gran = N_TILES * CH * SB
  e_pad = ((E + gran - 1) // gran) * gran
  padn = e_pad - E
  if padn:
    junk = JUNK_ROW + (jnp.arange(padn, dtype=jnp.int32) % (N_PAD - N_NODES))
    src = jnp.concatenate([src, junk])
    dst = jnp.concatenate([dst, junk])
  src2 = src.reshape(-1, CH)
  dst2 = dst.reshape(-1, CH)
  nch = e_pad // (N_TILES * CH)

  aggd = _sc_deg(nch)(dst2)
  y0 = _tc_matmul(features, W0)
  p0 = _sc_agg(128, nch)(y0, src2, dst2)
  y1 = _combine_mm(eps[0], y0, p0, aggd, b0, W1)
  p1 = _sc_agg(128, nch)(y1, src2, dst2)
  y2 = _combine_mm(eps[1], y1, p1, aggd, b1, W2)
  p2 = _sc_agg(128, nch)(y2, src2, dst2)
  W3p = jnp.pad(W3, ((0, 0), (0, 8)))
  y3 = _combine_mm(eps[2], y2, p2, aggd, b2, W3p)
  p3 = _sc_agg(48, nch)(y3, src2, dst2)
  out48 = _combine_final(eps[3], y3, p3, aggd, jnp.pad(b3, (0, 8)))
  return out48[:N_NODES, :40]


# speedup vs baseline: 1.1347x; 1.0045x over previous
import functools

import jax
import jax.numpy as jnp
from jax import lax
from jax.experimental import pallas as pl
from jax.experimental.pallas import tpu as pltpu
from jax.experimental.pallas import tpu_sc as plsc

N_NODES = 10000
N_PAD = 10240
JUNK_ROW = N_NODES
BLK = 5120
CH = 128
NB = 2
SB = 40
N_TILES = 32
RPT = N_PAD // 16

_MESH = dict(core_axis_name="c", subcore_axis_name="s")
_SC_PARAMS = dict(
    compiler_params=pltpu.CompilerParams(use_tc_tiling_on_sc=False))


def _fill(ref, rows, width, vec):
  for r in range(rows):
    for q in range(width // 16):
      ref[r, pl.ds(q * 16, 16)] = vec


def _sc_deg(nch):
  @functools.partial(
      pl.kernel,
      mesh=plsc.VectorSubcoreMesh(**_MESH),
      out_type=jax.ShapeDtypeStruct((2, N_PAD, 16), jnp.float32),
      scratch_types=[
          pltpu.VMEM_SHARED((N_PAD, 16), jnp.float32),
          pltpu.VMEM((nch, CH), jnp.int32),
          pltpu.VMEM((CH, 16), jnp.float32),
          pltpu.VMEM((64, 16), jnp.float32),
          pltpu.SemaphoreType.DMA,
      ],
      **_SC_PARAMS,
  )
  def k(dst_hbm, out_hbm, acc, dst_v, ones_v, zblk, sem):
    c = lax.axis_index("c")
    s = lax.axis_index("s")
    wid = s * 2 + c
    _fill(ones_v, CH, 16, jnp.ones((16,), jnp.float32))
    _fill(zblk, 64, 16, jnp.zeros((16,), jnp.float32))
    row0 = s * RPT

    def zbody(j, carry):
      pltpu.sync_copy(zblk, acc.at[pl.ds(row0 + j * 64, 64)])
      return carry

    lax.fori_loop(0, RPT // 64, zbody, 0)
    pltpu.sync_copy(dst_hbm.at[pl.ds(wid * nch, nch)], dst_v)
    plsc.subcore_barrier()

    def group(g, carry):
      j0 = g * 4
      hs = [pltpu.async_copy(ones_v, acc.at[dst_v.at[j0 + b]], sem, add=True)
            for b in range(4)]
      for h in hs:
        h.wait()
      return carry

    lax.fori_loop(0, nch // 4, group, 0)
    plsc.subcore_barrier()
    pltpu.sync_copy(acc.at[pl.ds(row0, RPT)], out_hbm.at[c, pl.ds(row0, RPT)])

  return k


def _sc_agg(D, nch):
  @functools.partial(
      pl.kernel,
      mesh=plsc.VectorSubcoreMesh(**_MESH),
      out_type=jax.ShapeDtypeStruct((2, N_PAD, D), jnp.float32),
      scratch_types=[
          pltpu.VMEM_SHARED((N_PAD, D), jnp.float32),
          pltpu.VMEM((SB, CH), jnp.int32),
          pltpu.VMEM((SB, CH), jnp.int32),
      ] + [pltpu.VMEM((CH, D), jnp.float32) for _ in range(NB)] + [
          pltpu.VMEM((16, D), jnp.float32),
          pltpu.SemaphoreType.DMA,
          pltpu.SemaphoreType.DMA,
      ],
      **_SC_PARAMS,
  )
  def k(y_hbm, src_hbm, dst_hbm, out_hbm, acc, src_v, dst_v,
        *bufs_zblk_sems):
    rows = list(bufs_zblk_sems[:NB])
    zblk, gsem, ssem = bufs_zblk_sems[NB:]
    c = lax.axis_index("c")
    s = lax.axis_index("s")
    wid = s * 2 + c
    _fill(zblk, 16, D, jnp.zeros((16,), jnp.float32))
    row0 = s * RPT

    def zbody(j, carry):
      pltpu.sync_copy(zblk, acc.at[pl.ds(row0 + j * 16, 16)])
      return carry

    lax.fori_loop(0, RPT // 16, zbody, 0)
    plsc.subcore_barrier()

    def sblock(sb, carry):
      chunk0 = wid * nch + sb * SB
      pltpu.sync_copy(src_hbm.at[pl.ds(chunk0, SB)], src_v)
      pltpu.sync_copy(dst_hbm.at[pl.ds(chunk0, SB)], dst_v)

      def group(g, carry2):
        j0 = g * NB
        ghs = [pltpu.async_copy(y_hbm.at[src_v.at[j0 + b]], rows[b], gsem)
               for b in range(NB)]
        shs = []
        for b in range(NB):
          ghs[b].wait()
          shs.append(pltpu.async_copy(rows[b], acc.at[dst_v.at[j0 + b]], ssem,
                                      add=True))
        for h in shs:
          h.wait()
        return carry2

      lax.fori_loop(0, SB // NB, group, 0)
      return carry

    lax.fori_loop(0, nch // SB, sblock, 0)
    plsc.subcore_barrier()
    pltpu.sync_copy(acc.at[pl.ds(row0, RPT)], out_hbm.at[c, pl.ds(row0, RPT)])

  return k


def _mm_body(x_ref, w_ref, o_ref):
  o_ref[...] = jnp.dot(x_ref[...], w_ref[...], preferred_element_type=jnp.float32)


def _tc_matmul(x, w):
  kdim = x.shape[1]
  dout = w.shape[1]
  return pl.pallas_call(
      _mm_body,
      grid=(N_PAD // BLK,),
      in_specs=[
          pl.BlockSpec((BLK, kdim), lambda i: (i, 0)),
          pl.BlockSpec((kdim, dout), lambda i: (0, 0)),
      ],
      out_specs=pl.BlockSpec((BLK, dout), lambda i: (i, 0)),
      out_shape=jax.ShapeDtypeStruct((N_PAD, dout), jnp.float32),
  )(x, w)


def _inv_deg(d0, d1):
  deg = jnp.maximum(d0[0] + d1[0], 1.0)
  return 1.0 / deg[:, 0:1]


def _comb_mm_body(eps_ref, y_ref, p0_ref, p1_ref, d0_ref, d1_ref, b_ref, w_ref,
                  o_ref):
  inv = _inv_deg(d0_ref[...], d1_ref[...])
  agg = (p0_ref[0] + p1_ref[0]) * inv
  h = (1.0 + eps_ref[0, 0]) * y_ref[...] + agg + b_ref[...]
  h = jnp.maximum(h, 0.0)
  o_ref[...] = jnp.dot(h, w_ref[...], preferred_element_type=jnp.float32)


def _comb_final_body(eps_ref, y_ref, p0_ref, p1_ref, d0_ref, d1_ref, b_ref,
                     o_ref):
  inv = _inv_deg(d0_ref[...], d1_ref[...])
  agg = (p0_ref[0] + p1_ref[0]) * inv
  o_ref[...] = (1.0 + eps_ref[0, 0]) * y_ref[...] + agg + b_ref[...]


_HALF = lambda half: (lambda i, h=half: (h, i, 0))
_DSPECS = [pl.BlockSpec((1, BLK, 16), _HALF(0)),
           pl.BlockSpec((1, BLK, 16), _HALF(1))]


def _combine_mm(eps_i, y, p, aggd, b, w):
  dy = y.shape[1]
  dout = w.shape[1]
  return pl.pallas_call(
      _comb_mm_body,
      grid=(N_PAD // BLK,),
      in_specs=[
          pl.BlockSpec(memory_space=pltpu.SMEM),
          pl.BlockSpec((BLK, dy), lambda i: (i, 0)),
          pl.BlockSpec((1, BLK, dy), _HALF(0)),
          pl.BlockSpec((1, BLK, dy), _HALF(1)),
          *_DSPECS,
          pl.BlockSpec((1, dy), lambda i: (0, 0)),
          pl.BlockSpec((dy, dout), lambda i: (0, 0)),
      ],
      out_specs=pl.BlockSpec((BLK, dout), lambda i: (i, 0)),
      out_shape=jax.ShapeDtypeStruct((N_PAD, dout), jnp.float32),
  )(eps_i.reshape(1, 1), y, p, p, aggd, aggd, b.reshape(1, dy), w)


def _combine_final(eps_i, y, p, aggd, b):
  dy = y.shape[1]
  return pl.pallas_call(
      _comb_final_body,
      grid=(N_PAD // BLK,),
      in_specs=[
          pl.BlockSpec(memory_space=pltpu.SMEM),
          pl.BlockSpec((BLK, dy), lambda i: (i, 0)),
          pl.BlockSpec((1, BLK, dy), _HALF(0)),
          pl.BlockSpec((1, BLK, dy), _HALF(1)),
          *_DSPECS,
          pl.BlockSpec((1, dy), lambda i: (0, 0)),
      ],
      out_specs=pl.BlockSpec((BLK, dy), lambda i: (i, 0)),
      out_shape=jax.ShapeDtypeStruct((N_PAD, dy), jnp.float32),
  )(eps_i.reshape(1, 1), y, p, p, aggd, aggd, b.reshape(1, dy))


def kernel(features, edge_index, W0, b0, W1, b1, W2, b2, W3, b3, eps):
  E = edge_index.shape[1]
  src, dst = edge_index[0], edge_index[1]
  gran = N_TILES * CH * SB
  e_pad = ((E + gran - 1) // gran) * gran
  padn = e_pad - E
  if padn:
    junk = JUNK_ROW + (jnp.arange(padn, dtype=jnp.int32) % (N_PAD - N_NODES))
    src = jnp.concatenate([src, junk])
    dst = jnp.concatenate([dst, junk])
  src2 = src.reshape(-1, CH)
  dst2 = dst.reshape(-1, CH)
  nch = e_pad // (N_TILES * CH)

  aggd = _sc_deg(nch)(dst2)
  y0 = _tc_matmul(features, W0)
  p0 = _sc_agg(128, nch)(y0, src2, dst2)
  y1 = _combine_mm(eps[0], y0, p0, aggd, b0, W1)
  p1 = _sc_agg(128, nch)(y1, src2, dst2)
  y2 = _combine_mm(eps[1], y1, p1, aggd, b1, W2)
  p2 = _sc_agg(128, nch)(y2, src2, dst2)
  W3p = jnp.pad(W3, ((0, 0), (0, 8)))
  y3 = _combine_mm(eps[2], y2, p2, aggd, b2, W3p)
  p3 = _sc_agg(48, nch)(y3, src2, dst2)
  out48 = _combine_final(eps[3], y3, p3, aggd, jnp.pad(b3, (0, 8)))
  return out48[:N_NODES, :40]
